# baseline (speedup 1.0000x reference)
import functools

import jax
import jax.numpy as jnp
from jax import lax
from jax.experimental import pallas as pl
from jax.experimental.pallas import tpu as pltpu
from jax.experimental.pallas import tpu_sc as plsc

NUM_CORES = 2
NUM_SUBCORES = 16
NW = NUM_CORES * NUM_SUBCORES
CH = 128


def _sc_tokenizer(B, NN, NCAT, D, n_cat_ch, n_num_ch):
    NTOK = NN + NCAT
    PITCH = (NTOK + 7) // 8 * 8
    mesh = plsc.VectorSubcoreMesh(
        core_axis_name="c", subcore_axis_name="s",
        num_cores=NUM_CORES, num_subcores=NUM_SUBCORES)

    @functools.partial(
        pl.kernel,
        out_type=jax.ShapeDtypeStruct((B, PITCH, D), jnp.float32),
        mesh=mesh,
        compiler_params=pltpu.CompilerParams(use_tc_tiling_on_sc=True),
        scratch_types=[
            pltpu.VMEM((n_cat_ch, CH), jnp.int32),
            pltpu.VMEM((n_cat_ch, CH), jnp.int32),
            pltpu.VMEM((n_num_ch, CH), jnp.int32),
            pltpu.VMEM((n_num_ch, CH), jnp.float32),
            pltpu.VMEM((NN, D), jnp.float32),
            pltpu.VMEM((NN, D), jnp.float32),
            pltpu.VMEM((2, 2, CH, D), jnp.float32),
            pltpu.SemaphoreType.DMA,
            pltpu.SemaphoreType.DMA,
        ],
    )
    def kern(tab_hbm, gidx_hbm, ocat_hbm, onum_hbm, xt_hbm, w_hbm, b_hbm,
             out3_hbm, gidx_v, ocat_v, onum_v, xt_v, w_v, bv_v, rows_v,
             gsem, ssem):
        out_hbm = out3_hbm.reshape(B * PITCH, D)
        wid = lax.axis_index("s") * NUM_CORES + lax.axis_index("c")
        pltpu.sync_copy(gidx_hbm.at[wid], gidx_v)
        pltpu.sync_copy(ocat_hbm.at[wid], ocat_v)
        pltpu.sync_copy(onum_hbm.at[wid], onum_v)
        pltpu.sync_copy(xt_hbm.at[wid], xt_v)
        pltpu.sync_copy(w_hbm, w_v)
        pltpu.sync_copy(b_hbm, bv_v)

        GRP = 2
        n_grp = n_cat_ch // GRP

        def wait_gather():
            pltpu.make_async_copy(
                tab_hbm.at[gidx_v.at[0]], rows_v.at[0, 0], gsem).wait()

        def wait_scatter():
            pltpu.make_async_copy(
                rows_v.at[0, 0], out_hbm.at[ocat_v.at[0]], ssem).wait()

        for t in range(GRP):
            pltpu.async_copy(tab_hbm.at[gidx_v.at[t]], rows_v.at[0, t], gsem)

        def cat_step(g, carry):
            bank = lax.rem(g, 2)
            for t in range(GRP):
                wait_gather()
            for t in range(GRP):
                pltpu.async_copy(rows_v.at[bank, t],
                                 out_hbm.at[ocat_v.at[g * GRP + t]], ssem)

            @pl.when(g >= 1)
            def _():
                for t in range(GRP):
                    wait_scatter()

            @pl.when(g + 1 < n_grp)
            def _():
                for t in range(GRP):
                    pltpu.async_copy(tab_hbm.at[gidx_v.at[(g + 1) * GRP + t]],
                                     rows_v.at[1 - bank, t], gsem)
            return carry
        lax.fori_loop(0, n_grp, cat_step, 0)
        for t in range(GRP):
            wait_scatter()

        bpc = n_num_ch // NN
        for f in range(NN):
            wv = [w_v[f, pl.ds(k * 16, 16)] for k in range(D // 16)]
            bv = [bv_v[f, pl.ds(k * 16, 16)] for k in range(D // 16)]

            def num_blk(bi, carry, f=f, wv=wv, bv=bv):
                jj = f * bpc + bi
                slot = lax.rem(jj, 2)

                @pl.when(jj >= 2)
                def _():
                    wait_scatter()

                def num_row(r0, c2):
                    x_v = xt_v[jj, pl.ds(r0 * 16, 16)]
                    for i in range(16):
                        x = x_v[i]
                        for k in range(D // 16):
                            rows_v[0, slot, r0 * 16 + i,
                                   pl.ds(k * 16, 16)] = x * wv[k] + bv[k]
                    return c2
                lax.fori_loop(0, CH // 16, num_row, 0)
                pltpu.async_copy(rows_v.at[0, slot],
                                 out_hbm.at[onum_v.at[jj]], ssem)
                return carry
            lax.fori_loop(0, bpc, num_blk, 0)
        for _ in range(2):
            wait_scatter()

    return kern


def kernel(x_num, x_cat, W_num, b_num, tables):
    B, NN = x_num.shape
    _, NCAT = x_cat.shape
    _, CARD, D = tables.shape
    NTOK = NN + NCAT
    bpw = B // NW
    n_cat_ch = bpw * NCAT // CH
    n_num_ch = bpw * NN // CH

    PITCH = (NTOK + 7) // 8 * 8
    tab_flat = tables.reshape(NCAT * CARD, D)
    foff = (jnp.arange(NCAT, dtype=jnp.int32) * CARD)[None, :]
    g_idx = (x_cat + foff).reshape(NW, n_cat_ch, CH)
    brow = jnp.arange(B, dtype=jnp.int32)[:, None] * PITCH
    o_cat = (brow + NN + jnp.arange(NCAT, dtype=jnp.int32)[None, :]).reshape(
        NW, n_cat_ch, CH)
    o_num = (jnp.arange(B, dtype=jnp.int32).reshape(NW, 1, bpw) * PITCH
             + jnp.arange(NN, dtype=jnp.int32).reshape(1, NN, 1)).reshape(
                 NW, n_num_ch, CH)
    x_t = x_num.reshape(NW, bpw, NN).transpose(0, 2, 1).reshape(
        NW, n_num_ch, CH)

    kern = _sc_tokenizer(B, NN, NCAT, D, n_cat_ch, n_num_ch)
    out = kern(tab_flat, g_idx, o_cat, o_num, x_t, W_num, b_num)
    return out[:, :NTOK, :]


# speedup vs baseline: 1.0006x; 1.0006x over previous
import functools

import jax
import jax.numpy as jnp
from jax import lax
from jax.experimental import pallas as pl
from jax.experimental.pallas import tpu as pltpu
from jax.experimental.pallas import tpu_sc as plsc

NUM_CORES = 2
NUM_SUBCORES = 16
NW = NUM_CORES * NUM_SUBCORES
CH = 128


def _sc_tokenizer(B, NN, NCAT, D, n_cat_ch, n_num_ch):
    NTOK = NN + NCAT
    PITCH = (NTOK + 7) // 8 * 8
    mesh = plsc.VectorSubcoreMesh(
        core_axis_name="c", subcore_axis_name="s",
        num_cores=NUM_CORES, num_subcores=NUM_SUBCORES)

    @functools.partial(
        pl.kernel,
        out_type=jax.ShapeDtypeStruct((B, PITCH, D), jnp.float32),
        mesh=mesh,
        compiler_params=pltpu.CompilerParams(
            use_tc_tiling_on_sc=True, needs_layout_passes=False),
        scratch_types=[
            pltpu.VMEM((n_cat_ch, CH), jnp.int32),
            pltpu.VMEM((n_cat_ch, CH), jnp.int32),
            pltpu.VMEM((n_num_ch, CH), jnp.int32),
            pltpu.VMEM((n_num_ch, CH), jnp.float32),
            pltpu.VMEM((NN, D), jnp.float32),
            pltpu.VMEM((NN, D), jnp.float32),
            pltpu.VMEM((2, 2, CH, D), jnp.float32),
            pltpu.SemaphoreType.DMA,
            pltpu.SemaphoreType.DMA,
        ],
    )
    def kern(tab_hbm, gidx_hbm, ocat_hbm, onum_hbm, xt_hbm, w_hbm, b_hbm,
             out3_hbm, gidx_v, ocat_v, onum_v, xt_v, w_v, bv_v, rows_v,
             gsem, ssem):
        out_hbm = out3_hbm.reshape(B * PITCH, D)
        wid = lax.axis_index("s") * NUM_CORES + lax.axis_index("c")
        pltpu.sync_copy(gidx_hbm.at[wid], gidx_v)
        pltpu.sync_copy(ocat_hbm.at[wid], ocat_v)
        pltpu.sync_copy(onum_hbm.at[wid], onum_v)
        pltpu.sync_copy(xt_hbm.at[wid], xt_v)
        pltpu.sync_copy(w_hbm, w_v)
        pltpu.sync_copy(b_hbm, bv_v)

        GRP = 2
        n_grp = n_cat_ch // GRP

        def wait_gather():
            pltpu.make_async_copy(
                tab_hbm.at[gidx_v.at[0]], rows_v.at[0, 0], gsem).wait()

        def wait_scatter():
            pltpu.make_async_copy(
                rows_v.at[0, 0], out_hbm.at[ocat_v.at[0]], ssem).wait()

        for t in range(GRP):
            pltpu.async_copy(tab_hbm.at[gidx_v.at[t]], rows_v.at[0, t], gsem)

        def cat_step(g, carry):
            bank = lax.rem(g, 2)
            for t in range(GRP):
                wait_gather()
            for t in range(GRP):
                pltpu.async_copy(rows_v.at[bank, t],
                                 out_hbm.at[ocat_v.at[g * GRP + t]], ssem)

            @pl.when(g >= 1)
            def _():
                for t in range(GRP):
                    wait_scatter()

            @pl.when(g + 1 < n_grp)
            def _():
                for t in range(GRP):
                    pltpu.async_copy(tab_hbm.at[gidx_v.at[(g + 1) * GRP + t]],
                                     rows_v.at[1 - bank, t], gsem)
            return carry
        lax.fori_loop(0, n_grp, cat_step, 0)
        for t in range(GRP):
            wait_scatter()

        bpc = n_num_ch // NN
        for f in range(NN):
            wv = [w_v[f, pl.ds(k * 16, 16)] for k in range(D // 16)]
            bv = [bv_v[f, pl.ds(k * 16, 16)] for k in range(D // 16)]

            def num_blk(bi, carry, f=f, wv=wv, bv=bv):
                jj = f * bpc + bi
                slot = lax.rem(jj, 2)

                @pl.when(jj >= 2)
                def _():
                    wait_scatter()

                def num_row(r0, c2):
                    x_v = xt_v[jj, pl.ds(r0 * 16, 16)]
                    for i in range(16):
                        x = x_v[i]
                        for k in range(D // 16):
                            rows_v[0, slot, r0 * 16 + i,
                                   pl.ds(k * 16, 16)] = x * wv[k] + bv[k]
                    return c2
                lax.fori_loop(0, CH // 16, num_row, 0)
                pltpu.async_copy(rows_v.at[0, slot],
                                 out_hbm.at[onum_v.at[jj]], ssem)
                return carry
            lax.fori_loop(0, bpc, num_blk, 0)
        for _ in range(2):
            wait_scatter()

    return kern


def kernel(x_num, x_cat, W_num, b_num, tables):
    B, NN = x_num.shape
    _, NCAT = x_cat.shape
    _, CARD, D = tables.shape
    NTOK = NN + NCAT
    bpw = B // NW
    n_cat_ch = bpw * NCAT // CH
    n_num_ch = bpw * NN // CH

    PITCH = (NTOK + 7) // 8 * 8
    tab_flat = tables.reshape(NCAT * CARD, D)
    foff = (jnp.arange(NCAT, dtype=jnp.int32) * CARD)[None, :]
    g_idx = (x_cat + foff).reshape(NW, n_cat_ch, CH)
    brow = jnp.arange(B, dtype=jnp.int32)[:, None] * PITCH
    o_cat = (brow + NN + jnp.arange(NCAT, dtype=jnp.int32)[None, :]).reshape(
        NW, n_cat_ch, CH)
    o_num = (jnp.arange(B, dtype=jnp.int32).reshape(NW, 1, bpw) * PITCH
             + jnp.arange(NN, dtype=jnp.int32).reshape(1, NN, 1)).reshape(
                 NW, n_num_ch, CH)
    x_t = x_num.reshape(NW, bpw, NN).transpose(0, 2, 1).reshape(
        NW, n_num_ch, CH)

    kern = _sc_tokenizer(B, NN, NCAT, D, n_cat_ch, n_num_ch)
    out = kern(tab_flat, g_idx, o_cat, o_num, x_t, W_num, b_num)
    return out[:, :NTOK, :]


# speedup vs baseline: 2.1550x; 2.1537x over previous
import functools

import jax
import jax.numpy as jnp
from jax import lax
from jax.experimental import pallas as pl
from jax.experimental.pallas import tpu as pltpu
from jax.experimental.pallas import tpu_sc as plsc

NUM_CORES = 2
NUM_SUBCORES = 16
NW = NUM_CORES * NUM_SUBCORES
CH = 128


def _sc_tokenizer(B, NN, NCAT, D, n_cat_ch):
    NTOK = NN + NCAT
    bpw = B // NW
    mesh = plsc.VectorSubcoreMesh(
        core_axis_name="c", subcore_axis_name="s",
        num_cores=NUM_CORES, num_subcores=NUM_SUBCORES)

    @functools.partial(
        pl.kernel,
        out_type=jax.ShapeDtypeStruct((NTOK * B, D), jnp.float32),
        mesh=mesh,
        scratch_types=[
            pltpu.VMEM((n_cat_ch, CH), jnp.int32),
            pltpu.VMEM((NN * bpw // CH, CH), jnp.float32),
            pltpu.VMEM((NN, D), jnp.float32),
            pltpu.VMEM((NN, D), jnp.float32),
            pltpu.VMEM((2, 2, CH, D), jnp.float32),
            pltpu.VMEM((2, CH, D), jnp.float32),
            pltpu.SemaphoreType.DMA,
            pltpu.SemaphoreType.DMA,
            pltpu.SemaphoreType.DMA,
        ],
    )
    def kern(tab_hbm, gidx_hbm, xt_hbm, w_hbm, b_hbm, out_hbm,
             gidx_v, xt_v, w_v, bv_v, rows_v, nrows_v,
             gsem, ssem, nsem):
        wid = lax.axis_index("s") * NUM_CORES + lax.axis_index("c")
        pltpu.sync_copy(gidx_hbm.at[wid], gidx_v)
        pltpu.sync_copy(xt_hbm.at[wid], xt_v)
        pltpu.sync_copy(w_hbm, w_v)
        pltpu.sync_copy(b_hbm, bv_v)

        GRP = 2
        n_grp = n_cat_ch // GRP
        bpc = bpw // CH
        nb = NN * bpc

        def wait_gather():
            pltpu.make_async_copy(
                tab_hbm.at[gidx_v.at[0]], rows_v.at[0, 0], gsem).wait()

        def wait_scatter():
            pltpu.make_async_copy(
                rows_v.at[0, 0], out_hbm.at[pl.ds(0, CH)], ssem).wait()

        def wait_num():
            pltpu.make_async_copy(
                nrows_v.at[0], out_hbm.at[pl.ds(0, CH)], nsem).wait()

        def num_block(j):
            slot = lax.rem(j, 2)

            @pl.when(j >= 2)
            def _():
                wait_num()

            f = lax.div(j, bpc)
            bi = lax.rem(j, bpc)
            wv = [w_v[f, pl.ds(k * 16, 16)] for k in range(D // 16)]
            bv = [bv_v[f, pl.ds(k * 16, 16)] for k in range(D // 16)]

            def num_row(r0, c2):
                x_v = xt_v[j, pl.ds(r0 * 16, 16)]
                for i in range(16):
                    x = x_v[i]
                    for k in range(D // 16):
                        nrows_v[slot, r0 * 16 + i,
                                pl.ds(k * 16, 16)] = x * wv[k] + bv[k]
                return c2
            lax.fori_loop(0, CH // 16, num_row, 0)
            dst = f * B + wid * bpw + bi * CH
            pltpu.async_copy(nrows_v.at[slot],
                             out_hbm.at[pl.ds(dst, CH)], nsem)

        for t in range(GRP):
            pltpu.async_copy(tab_hbm.at[gidx_v.at[t]], rows_v.at[0, t], gsem)

        def cat_step(g, carry):
            bank = lax.rem(g, 2)
            for t in range(GRP):
                wait_gather()
            for t in range(GRP):
                c = g * GRP + t
                cdst = ((NN + lax.div(c, bpc)) * B + wid * bpw
                        + lax.rem(c, bpc) * CH)
                pltpu.async_copy(rows_v.at[bank, t],
                                 out_hbm.at[pl.ds(cdst, CH)], ssem)

            @pl.when(g >= 1)
            def _():
                for t in range(GRP):
                    wait_scatter()

            @pl.when(g + 1 < n_grp)
            def _():
                for t in range(GRP):
                    pltpu.async_copy(tab_hbm.at[gidx_v.at[(g + 1) * GRP + t]],
                                     rows_v.at[1 - bank, t], gsem)

            @pl.when(g < nb)
            def _():
                num_block(g)
            return carry
        lax.fori_loop(0, n_grp, cat_step, 0)
        for t in range(GRP):
            wait_scatter()

        def tail_step(j, carry):
            num_block(j)
            return carry
        lax.fori_loop(n_grp, nb, tail_step, 0)
        for _ in range(min(2, nb)):
            wait_num()

    return kern


def kernel(x_num, x_cat, W_num, b_num, tables):
    B, NN = x_num.shape
    _, NCAT = x_cat.shape
    _, CARD, D = tables.shape
    NTOK = NN + NCAT
    bpw = B // NW
    n_cat_ch = bpw * NCAT // CH

    tab_flat = tables.reshape(NCAT * CARD, D)
    foff = (jnp.arange(NCAT, dtype=jnp.int32) * CARD)[None, :]
    g_idx = (x_cat + foff).reshape(NW, bpw, NCAT).transpose(0, 2, 1).reshape(
        NW, n_cat_ch, CH)
    x_t = x_num.reshape(NW, bpw, NN).transpose(0, 2, 1).reshape(
        NW, NN * bpw // CH, CH)

    kern = _sc_tokenizer(B, NN, NCAT, D, n_cat_ch)
    out = kern(tab_flat, g_idx, x_t, W_num, b_num)
    return jnp.transpose(out.reshape(NTOK, B, D), (1, 0, 2))


# speedup vs baseline: 2.1564x; 1.0007x over previous
import functools

import jax
import jax.numpy as jnp
from jax import lax
from jax.experimental import pallas as pl
from jax.experimental.pallas import tpu as pltpu
from jax.experimental.pallas import tpu_sc as plsc

NUM_CORES = 2
NUM_SUBCORES = 16
NW = NUM_CORES * NUM_SUBCORES
CH = 128


def _sc_tokenizer(B, NN, NCAT, D, n_cat_ch):
    NTOK = NN + NCAT
    bpw = B // NW
    mesh = plsc.VectorSubcoreMesh(
        core_axis_name="c", subcore_axis_name="s",
        num_cores=NUM_CORES, num_subcores=NUM_SUBCORES)

    @functools.partial(
        pl.kernel,
        out_type=jax.ShapeDtypeStruct((NTOK * B, D), jnp.float32),
        mesh=mesh,
        scratch_types=[
            pltpu.VMEM((n_cat_ch, CH), jnp.int32),
            pltpu.VMEM((NN * bpw // CH, CH), jnp.float32),
            pltpu.VMEM((NN, D), jnp.float32),
            pltpu.VMEM((NN, D), jnp.float32),
            pltpu.VMEM((2, 2, CH, D), jnp.float32),
            pltpu.VMEM((2, CH, D), jnp.float32),
            pltpu.SemaphoreType.DMA,
            pltpu.SemaphoreType.DMA,
            pltpu.SemaphoreType.DMA,
        ],
    )
    def kern(tab_hbm, gidx_hbm, xt_hbm, w_hbm, b_hbm, out_hbm,
             gidx_v, xt_v, w_v, bv_v, rows_v, nrows_v,
             gsem, ssem, nsem):
        wid = lax.axis_index("s") * NUM_CORES + lax.axis_index("c")
        pltpu.sync_copy(gidx_hbm.at[wid], gidx_v)
        for t in range(2):
            pltpu.async_copy(tab_hbm.at[gidx_v.at[t]], rows_v.at[0, t], gsem)
        pltpu.sync_copy(xt_hbm.at[wid], xt_v)
        pltpu.sync_copy(w_hbm, w_v)
        pltpu.sync_copy(b_hbm, bv_v)

        GRP = 2
        n_grp = n_cat_ch // GRP
        bpc = bpw // CH
        nb = NN * bpc

        def wait_gather():
            pltpu.make_async_copy(
                tab_hbm.at[gidx_v.at[0]], rows_v.at[0, 0], gsem).wait()

        def wait_scatter():
            pltpu.make_async_copy(
                rows_v.at[0, 0], out_hbm.at[pl.ds(0, CH)], ssem).wait()

        def wait_num():
            pltpu.make_async_copy(
                nrows_v.at[0], out_hbm.at[pl.ds(0, CH)], nsem).wait()

        def num_block(j):
            slot = lax.rem(j, 2)

            @pl.when(j >= 2)
            def _():
                wait_num()

            f = lax.div(j, bpc)
            bi = lax.rem(j, bpc)
            wv = [w_v[f, pl.ds(k * 16, 16)] for k in range(D // 16)]
            bv = [bv_v[f, pl.ds(k * 16, 16)] for k in range(D // 16)]

            def num_row(r0, c2):
                x_v = xt_v[j, pl.ds(r0 * 16, 16)]
                for i in range(16):
                    x = x_v[i]
                    for k in range(D // 16):
                        nrows_v[slot, r0 * 16 + i,
                                pl.ds(k * 16, 16)] = x * wv[k] + bv[k]
                return c2
            lax.fori_loop(0, CH // 16, num_row, 0)
            dst = f * B + wid * bpw + bi * CH
            pltpu.async_copy(nrows_v.at[slot],
                             out_hbm.at[pl.ds(dst, CH)], nsem)

        def cat_step(g, carry):
            bank = lax.rem(g, 2)
            for t in range(GRP):
                wait_gather()
            for t in range(GRP):
                c = g * GRP + t
                cdst = ((NN + lax.div(c, bpc)) * B + wid * bpw
                        + lax.rem(c, bpc) * CH)
                pltpu.async_copy(rows_v.at[bank, t],
                                 out_hbm.at[pl.ds(cdst, CH)], ssem)

            @pl.when(g >= 1)
            def _():
                for t in range(GRP):
                    wait_scatter()

            @pl.when(g + 1 < n_grp)
            def _():
                for t in range(GRP):
                    pltpu.async_copy(tab_hbm.at[gidx_v.at[(g + 1) * GRP + t]],
                                     rows_v.at[1 - bank, t], gsem)

            @pl.when(g < nb)
            def _():
                num_block(g)
            return carry
        lax.fori_loop(0, n_grp, cat_step, 0)
        for t in range(GRP):
            wait_scatter()

        def tail_step(j, carry):
            num_block(j)
            return carry
        lax.fori_loop(n_grp, nb, tail_step, 0)
        for _ in range(min(2, nb)):
            wait_num()

    return kern


def kernel(x_num, x_cat, W_num, b_num, tables):
    B, NN = x_num.shape
    _, NCAT = x_cat.shape
    _, CARD, D = tables.shape
    NTOK = NN + NCAT
    bpw = B // NW
    n_cat_ch = bpw * NCAT // CH

    tab_flat = tables.reshape(NCAT * CARD, D)
    foff = (jnp.arange(NCAT, dtype=jnp.int32) * CARD)[None, :]
    g_idx = (x_cat + foff).reshape(NW, bpw, NCAT).transpose(0, 2, 1).reshape(
        NW, n_cat_ch, CH)
    x_t = x_num.reshape(NW, bpw, NN).transpose(0, 2, 1).reshape(
        NW, NN * bpw // CH, CH)

    kern = _sc_tokenizer(B, NN, NCAT, D, n_cat_ch)
    out = kern(tab_flat, g_idx, x_t, W_num, b_num)
    return jnp.transpose(out.reshape(NTOK, B, D), (1, 0, 2))
